# baseline (speedup 1.0000x reference)
import functools

import jax
import jax.numpy as jnp
import numpy as np
from jax import lax
from jax.experimental import pallas as pl
from jax.experimental.pallas import tpu as pltpu
from jax.experimental.pallas import tpu_sc as plsc

N = 10000
EMB = 128
CAT = 768
NBINS = 100
BSZ = 1024
NNEG = 4

RB = 400
NB = N // RB
NGATHER = BSZ * (2 + NNEG)

_F32 = jnp.float32


def _item_body(f_ref, p_ref, wc2_ref, wc3_ref, we_ref, bc2_ref, bc3_ref,
               be_ref, ep_ref, item_ref, s_ref, wf_s, pf_s, b_s, acc_s):
    i = pl.program_id(0)

    @pl.when(i == 0)
    def _():
        we1 = we_ref[0:EMB, :]
        we2 = we_ref[EMB:2 * EMB, :]
        we3 = we_ref[2 * EMB:3 * EMB, :]
        wf_s[0:CAT, :] = jnp.dot(wc2_ref[...], we1, preferred_element_type=_F32)
        wf_s[CAT:2 * CAT, :] = jnp.dot(wc3_ref[...], we2, preferred_element_type=_F32)
        pf_s[...] = jnp.dot(ep_ref[...], we3, preferred_element_type=_F32)
        b_s[...] = (jnp.dot(bc2_ref[...], we1, preferred_element_type=_F32)
                    + jnp.dot(bc3_ref[...], we2, preferred_element_type=_F32)
                    + be_ref[...])
        acc_s[...] = jnp.zeros_like(acc_s)

    onehot = (p_ref[...] == lax.broadcasted_iota(jnp.int32, (RB, 128), 1)).astype(_F32)
    x = jnp.dot(f_ref[...], wf_s[...], preferred_element_type=_F32)
    x = x + jnp.dot(onehot, pf_s[...], preferred_element_type=_F32) + b_s[...]
    it = jnp.maximum(x, 0.0)
    item_ref[...] = it
    acc_s[...] += jnp.sum(it, axis=0, keepdims=True)

    @pl.when(i == NB - 1)
    def _():
        s_ref[...] = acc_s[...]


def _item_call(features, pricei, wc2, wc3, we, bc2, bc3, be, ep_pad):
    return pl.pallas_call(
        _item_body,
        grid=(NB,),
        in_specs=[
            pl.BlockSpec((RB, 2 * CAT), lambda i: (i, 0)),
            pl.BlockSpec((RB, 1), lambda i: (i, 0)),
            pl.BlockSpec((CAT, EMB), lambda i: (0, 0)),
            pl.BlockSpec((CAT, EMB), lambda i: (0, 0)),
            pl.BlockSpec((3 * EMB, EMB), lambda i: (0, 0)),
            pl.BlockSpec((1, EMB), lambda i: (0, 0)),
            pl.BlockSpec((1, EMB), lambda i: (0, 0)),
            pl.BlockSpec((1, EMB), lambda i: (0, 0)),
            pl.BlockSpec((128, EMB), lambda i: (0, 0)),
        ],
        out_specs=[
            pl.BlockSpec((RB, EMB), lambda i: (i, 0)),
            pl.BlockSpec((1, EMB), lambda i: (0, 0)),
        ],
        out_shape=[
            jax.ShapeDtypeStruct((N, EMB), _F32),
            jax.ShapeDtypeStruct((1, EMB), _F32),
        ],
        scratch_shapes=[
            pltpu.VMEM((2 * CAT, EMB), _F32),
            pltpu.VMEM((128, EMB), _F32),
            pltpu.VMEM((1, EMB), _F32),
            pltpu.VMEM((1, EMB), _F32),
        ],
    )(features, pricei, wc2, wc3, we, bc2, bc3, be, ep_pad)


def _spmm_body(adj_ref, x_ref, t_ref, s_ref, acc_s):
    i = pl.program_id(0)

    @pl.when(i == 0)
    def _():
        acc_s[...] = jnp.zeros_like(acc_s)

    t = jnp.dot(adj_ref[...].astype(jnp.bfloat16), x_ref[...].astype(jnp.bfloat16),
                preferred_element_type=_F32)
    t_ref[...] = t
    acc_s[...] += jnp.sum(t, axis=0, keepdims=True)

    @pl.when(i == NB - 1)
    def _():
        s_ref[...] = acc_s[...]


def _spmm_call(adj, x):
    return pl.pallas_call(
        _spmm_body,
        grid=(NB,),
        in_specs=[
            pl.BlockSpec((RB, N), lambda i: (i, 0)),
            pl.BlockSpec((N, EMB), lambda i: (0, 0)),
        ],
        out_specs=[
            pl.BlockSpec((RB, EMB), lambda i: (i, 0)),
            pl.BlockSpec((1, EMB), lambda i: (0, 0)),
        ],
        out_shape=[
            jax.ShapeDtypeStruct((N, EMB), _F32),
            jax.ShapeDtypeStruct((1, EMB), _F32),
        ],
        scratch_shapes=[pltpu.VMEM((1, EMB), _F32)],
    )(adj, x)


W2 = 1024
NK2 = (N + W2 - 1) // W2
LASTW2 = N - (NK2 - 1) * W2
NPAD = NK2 * W2
RB2 = 2000
NG2 = N // RB2
GPB = RB2 // RB
_BF = jnp.bfloat16


def _p1_width(i2):
    return (RB2 * i2 // W2) * W2


def _p1_body(adj_ref, x_ref, t1_ref, t2p_ref, s1_ref, t1b_ref, s1_s):
    i = pl.program_id(0)
    acc1 = jnp.dot(adj_ref[...].astype(_BF), x_ref[...].astype(_BF),
                   preferred_element_type=_F32)
    t1_ref[...] = acc1
    t1b_ref[pl.ds(i * RB, RB), :] = acc1.astype(_BF)
    cs = jnp.sum(acc1, axis=0, keepdims=True)
    s1_s[...] = jnp.where(i == 0, cs, s1_s[...] + cs)

    @pl.when(i == NB - 1)
    def _():
        s1_ref[...] = s1_s[...]
        t1b_ref[N:NPAD, :] = jnp.zeros((NPAD - N, EMB), _BF)

    i2 = i // GPB

    @pl.when(i2 == 0)
    def _():
        t2p_ref[...] = jnp.zeros((RB, EMB), _F32)

    for gidx in range(1, NG2):
        wdt = _p1_width(gidx)

        @pl.when(i2 == gidx)
        def _():
            t2p_ref[...] = jnp.dot(adj_ref[:, 0:wdt].astype(_BF),
                                   t1b_ref[0:wdt, :],
                                   preferred_element_type=_F32)


def _p1_call(adj, x):
    return pl.pallas_call(
        _p1_body,
        grid=(NB,),
        in_specs=[
            pl.BlockSpec((RB, N), lambda i: (i, 0)),
            pl.BlockSpec((N, EMB), lambda i: (0, 0)),
        ],
        out_specs=[
            pl.BlockSpec((RB, EMB), lambda i: (i, 0)),
            pl.BlockSpec((RB, EMB), lambda i: (i, 0)),
            pl.BlockSpec((1, EMB), lambda i: (0, 0)),
            pl.BlockSpec((NPAD, EMB), lambda i: (0, 0)),
        ],
        out_shape=[
            jax.ShapeDtypeStruct((N, EMB), _F32),
            jax.ShapeDtypeStruct((N, EMB), _F32),
            jax.ShapeDtypeStruct((1, EMB), _F32),
            jax.ShapeDtypeStruct((NPAD, EMB), _BF),
        ],
        scratch_shapes=[
            pltpu.VMEM((1, EMB), _F32),
        ],
    )(adj, x)


def _build_p2_sched():
    steps = []
    for i2 in range(NG2):
        ks = list(range(_p1_width(i2) // W2, NK2))
        for n, k in enumerate(ks):
            fl = ((1 if n == 0 else 0)
                  | (2 if n == len(ks) - 1 else 0)
                  | (4 if k == NK2 - 1 else 0))
            steps.append((i2, k, fl))
    arr = np.asarray(steps, dtype=np.int32)
    return arr[:, 0], arr[:, 1], arr[:, 2]


_I2A, _K2A, _FL2 = _build_p2_sched()
_NP2 = len(_I2A)


def _p2_body(i2_ref, k_ref, fl_ref, adj_ref, t1p_ref, t2in_ref,
             t2_ref, s2_ref, acc_s, s2_s):
    g = pl.program_id(0)
    k = k_ref[g]
    fl = fl_ref[g]

    @pl.when(g == 0)
    def _():
        s2_s[...] = jnp.zeros_like(s2_s)

    @pl.when((fl & 1) != 0)
    def _():
        acc_s[...] = t2in_ref[...]

    @pl.when((fl & 4) == 0)
    def _():
        acc_s[...] += jnp.dot(adj_ref[...].astype(_BF),
                              t1p_ref[pl.ds(k * W2, W2), :],
                              preferred_element_type=_F32)

    @pl.when((fl & 4) != 0)
    def _():
        am = jnp.where(lax.broadcasted_iota(jnp.int32, (RB2, W2), 1) < LASTW2,
                       adj_ref[...], 0.0)
        acc_s[...] += jnp.dot(am.astype(_BF), t1p_ref[pl.ds(k * W2, W2), :],
                              preferred_element_type=_F32)

    @pl.when((fl & 2) != 0)
    def _():
        t2_ref[...] = acc_s[...]
        s2_s[...] += jnp.sum(acc_s[...], axis=0, keepdims=True)

    @pl.when(g == _NP2 - 1)
    def _():
        s2_ref[...] = s2_s[...]


def _p2_call(adj, t1pad, t2p):
    grid_spec = pltpu.PrefetchScalarGridSpec(
        num_scalar_prefetch=3,
        grid=(_NP2,),
        in_specs=[
            pl.BlockSpec((RB2, W2), lambda g, ia, ka, fl: (ia[g], ka[g])),
            pl.BlockSpec((NPAD, EMB), lambda g, ia, ka, fl: (0, 0)),
            pl.BlockSpec((RB2, EMB), lambda g, ia, ka, fl: (ia[g], 0)),
        ],
        out_specs=[
            pl.BlockSpec((RB2, EMB), lambda g, ia, ka, fl: (ia[g], 0)),
            pl.BlockSpec((1, EMB), lambda g, ia, ka, fl: (0, 0)),
        ],
        scratch_shapes=[
            pltpu.VMEM((RB2, EMB), _F32),
            pltpu.VMEM((1, EMB), _F32),
        ],
    )
    return pl.pallas_call(
        _p2_body,
        grid_spec=grid_spec,
        out_shape=[
            jax.ShapeDtypeStruct((N, EMB), _F32),
            jax.ShapeDtypeStruct((1, EMB), _F32),
        ],
    )(jnp.asarray(_I2A), jnp.asarray(_K2A), jnp.asarray(_FL2), adj, t1pad, t2p)


def _gc_body(t1_ref, t2_ref, it_ref, s1_ref, s2_ref, si_ref,
             wl_ref, wm_ref, g1_ref, be1_ref, g2_ref, be2_ref,
             wct_ref, wcb_ref, bcat_ref,
             a_ref, b_ref, c_ref, mu_s, glp_s, gmp_s):
    i = pl.program_id(0)
    ninv = 1.0 / N

    @pl.when(i == 0)
    def _():
        mu_s[0:1, :] = 0.5 * (s1_ref[...] + si_ref[...]) * ninv
        mu_s[1:2, :] = 0.5 * (s2_ref[...] - si_ref[...]) * ninv
        glp_s[...] = jnp.zeros_like(glp_s)
        gmp_s[...] = jnp.zeros_like(gmp_s)

    t1 = t1_ref[...]
    t2 = t2_ref[...]
    it = it_ref[...]
    lpc = 0.5 * (t1 + it) - mu_s[0:1, :]
    mpc = 0.5 * (t2 - it) - mu_s[1:2, :]
    dn = (((0,), (0,)), ((), ()))
    lpc = lpc.astype(jnp.bfloat16)
    mpc = mpc.astype(jnp.bfloat16)
    glp_s[...] += lax.dot_general(lpc, lpc, dn, preferred_element_type=_F32)
    gmp_s[...] += lax.dot_general(mpc, mpc, dn, preferred_element_type=_F32)

    @pl.when(i == NB - 1)
    def _():
        mu_lp = mu_s[0:1, :]
        mu_mp = mu_s[1:2, :]
        wl = wl_ref[...]
        wm = wm_ref[...]
        m1 = jnp.dot(mu_lp, wl, preferred_element_type=_F32)
        var1 = jnp.sum(jnp.dot(glp_s[...], wl, preferred_element_type=_F32) * wl,
                       axis=0, keepdims=True) * ninv
        a1 = g1_ref[...] / jnp.sqrt(var1 + 1e-5)
        m2 = jnp.dot(mu_mp, wm, preferred_element_type=_F32)
        var2 = jnp.sum(jnp.dot(gmp_s[...], wm, preferred_element_type=_F32) * wm,
                       axis=0, keepdims=True) * ninv
        a2 = g2_ref[...] / jnp.sqrt(var2 + 1e-5)
        wct = wct_ref[...]
        wcb = wcb_ref[...]
        a_ref[...] = 0.5 * jnp.dot(wl * a1, wct, preferred_element_type=_F32)
        b_ref[...] = 0.5 * jnp.dot(wm * a2, wcb, preferred_element_type=_F32)
        c_ref[...] = (jnp.dot(be1_ref[...] - m1 * a1, wct, preferred_element_type=_F32)
                      + jnp.dot(be2_ref[...] - m2 * a2, wcb, preferred_element_type=_F32)
                      + bcat_ref[...])


def _gc_call(t1, t2, item, s1, s2, si, wl, wm, g1, be1, g2, be2, wct, wcb, bcat):
    blk = pl.BlockSpec((RB, EMB), lambda i: (i, 0))
    vec = pl.BlockSpec((1, EMB), lambda i: (0, 0))
    small = pl.BlockSpec((EMB, EMB), lambda i: (0, 0))
    return pl.pallas_call(
        _gc_body,
        grid=(NB,),
        in_specs=[blk, blk, blk, vec, vec, vec,
                  small, small, vec, vec, vec, vec, small, small, vec],
        out_specs=[small, small, vec],
        out_shape=[
            jax.ShapeDtypeStruct((EMB, EMB), _F32),
            jax.ShapeDtypeStruct((EMB, EMB), _F32),
            jax.ShapeDtypeStruct((1, EMB), _F32),
        ],
        scratch_shapes=[
            pltpu.VMEM((2, EMB), _F32),
            pltpu.VMEM((EMB, EMB), _F32),
            pltpu.VMEM((EMB, EMB), _F32),
        ],
    )(t1, t2, item, s1, s2, si, wl, wm, g1, be1, g2, be2, wct, wcb, bcat)


_SC_INFO = plsc.get_sparse_core_info()
_NW = _SC_INFO.num_cores * _SC_INFO.num_subcores
_BPW = NGATHER // _NW
_CH = 96


def _sc_gather_body(t1_hbm, t2_hbm, it_hbm, idx_hbm, o1, o2, o3,
                    idx_v, rows_v, sem):
    wid = lax.axis_index("s") * _SC_INFO.num_cores + lax.axis_index("c")
    base = wid * _BPW
    for ci in range(_BPW // _CH):
        off = base + ci * _CH
        pltpu.sync_copy(idx_hbm.at[pl.ds(off, _CH)], idx_v)
        for tab, out in ((t1_hbm, o1), (t2_hbm, o2), (it_hbm, o3)):
            pltpu.async_copy(tab.at[idx_v], rows_v, sem).wait()
            pltpu.sync_copy(rows_v, out.at[pl.ds(off, _CH)])


_sc_gather = functools.partial(
    pl.kernel,
    mesh=plsc.VectorSubcoreMesh(core_axis_name="c", subcore_axis_name="s"),
    out_type=[jax.ShapeDtypeStruct((NGATHER, EMB), _F32)] * 3,
    scratch_types=[
        pltpu.VMEM((_CH,), jnp.int32),
        pltpu.VMEM((_CH, EMB), _F32),
        pltpu.SemaphoreType.DMA,
    ],
)(_sc_gather_body)


def _loss_body(t1g_ref, t2g_ref, ig_ref, a_ref, b_ref, c_ref, out_ref):
    ig = ig_ref[...]
    og = (jnp.dot(t1g_ref[...] + ig, a_ref[...], preferred_element_type=_F32)
          + jnp.dot(t2g_ref[...] - ig, b_ref[...], preferred_element_type=_F32)
          + c_ref[...])
    key = og[0:BSZ]
    pos = og[BSZ:2 * BSZ]
    ps = jnp.sum(key * pos, axis=1, keepdims=True)
    acc = jnp.zeros((1, 1), _F32)
    for k in range(NNEG):
        ns = jnp.sum(key * og[(2 + k) * BSZ:(3 + k) * BSZ], axis=1, keepdims=True)
        x = ps - ns
        sig = 1.0 / (1.0 + jnp.exp(-x))
        acc = acc + jnp.sum(jnp.log(sig + 1e-9))
    out_ref[...] = -acc / (BSZ * NNEG)


def _loss_call(t1g, t2g, ig, a, b, c):
    return pl.pallas_call(
        _loss_body,
        out_shape=jax.ShapeDtypeStruct((1, 1), _F32),
    )(t1g, t2g, ig, a, b, c)


def kernel(features, price, adj, train_set, W_cid2, b_cid2, W_cid3, b_cid3,
           emb_price, W_emb, b_emb, W_low, W_mid, g1, be1, g2, be2,
           W_cat, b_cat):
    pricei = price.reshape(N, 1)
    ep_pad = jnp.pad(emb_price, ((0, 128 - NBINS), (0, 0)))
    r = lambda v: v.reshape(1, EMB)

    item, s_it = _item_call(features, pricei, W_cid2, W_cid3, W_emb,
                            r(b_cid2), r(b_cid3), r(b_emb), ep_pad)
    t1, t2p, s1, t1b = _p1_call(adj, item)
    t2, s2 = _p2_call(adj, t1b, t2p)
    a, b, c = _gc_call(t1, t2, item, s1, s2, s_it, W_low, W_mid,
                       r(g1), r(be1), r(g2), r(be2),
                       W_cat[:EMB], W_cat[EMB:], r(b_cat))
    idx = train_set.T.reshape(-1)
    t1g, t2g, ig = _sc_gather(t1, t2, item, idx)
    loss = _loss_call(t1g, t2g, ig, a, b, c)
    return loss.reshape(())


# speedup vs baseline: 1.0015x; 1.0015x over previous
import functools

import jax
import jax.numpy as jnp
import numpy as np
from jax import lax
from jax.experimental import pallas as pl
from jax.experimental.pallas import tpu as pltpu
from jax.experimental.pallas import tpu_sc as plsc

N = 10000
EMB = 128
CAT = 768
NBINS = 100
BSZ = 1024
NNEG = 4

RB = 400
NB = N // RB
NGATHER = BSZ * (2 + NNEG)

_F32 = jnp.float32


def _item_body(f_ref, p_ref, wc2_ref, wc3_ref, we_ref, bc2_ref, bc3_ref,
               be_ref, ep_ref, item_ref, s_ref, wf_s, pf_s, b_s, acc_s):
    i = pl.program_id(0)

    @pl.when(i == 0)
    def _():
        we1 = we_ref[0:EMB, :]
        we2 = we_ref[EMB:2 * EMB, :]
        we3 = we_ref[2 * EMB:3 * EMB, :]
        wf_s[0:CAT, :] = jnp.dot(wc2_ref[...], we1, preferred_element_type=_F32)
        wf_s[CAT:2 * CAT, :] = jnp.dot(wc3_ref[...], we2, preferred_element_type=_F32)
        pf_s[...] = jnp.dot(ep_ref[...], we3, preferred_element_type=_F32)
        b_s[...] = (jnp.dot(bc2_ref[...], we1, preferred_element_type=_F32)
                    + jnp.dot(bc3_ref[...], we2, preferred_element_type=_F32)
                    + be_ref[...])
        acc_s[...] = jnp.zeros_like(acc_s)

    onehot = (p_ref[...] == lax.broadcasted_iota(jnp.int32, (RB, 128), 1)).astype(_F32)
    x = jnp.dot(f_ref[...], wf_s[...], preferred_element_type=_F32)
    x = x + jnp.dot(onehot, pf_s[...], preferred_element_type=_F32) + b_s[...]
    it = jnp.maximum(x, 0.0)
    item_ref[...] = it
    acc_s[...] += jnp.sum(it, axis=0, keepdims=True)

    @pl.when(i == NB - 1)
    def _():
        s_ref[...] = acc_s[...]


def _item_call(features, pricei, wc2, wc3, we, bc2, bc3, be, ep_pad):
    return pl.pallas_call(
        _item_body,
        grid=(NB,),
        in_specs=[
            pl.BlockSpec((RB, 2 * CAT), lambda i: (i, 0)),
            pl.BlockSpec((RB, 1), lambda i: (i, 0)),
            pl.BlockSpec((CAT, EMB), lambda i: (0, 0)),
            pl.BlockSpec((CAT, EMB), lambda i: (0, 0)),
            pl.BlockSpec((3 * EMB, EMB), lambda i: (0, 0)),
            pl.BlockSpec((1, EMB), lambda i: (0, 0)),
            pl.BlockSpec((1, EMB), lambda i: (0, 0)),
            pl.BlockSpec((1, EMB), lambda i: (0, 0)),
            pl.BlockSpec((128, EMB), lambda i: (0, 0)),
        ],
        out_specs=[
            pl.BlockSpec((RB, EMB), lambda i: (i, 0)),
            pl.BlockSpec((1, EMB), lambda i: (0, 0)),
        ],
        out_shape=[
            jax.ShapeDtypeStruct((N, EMB), _F32),
            jax.ShapeDtypeStruct((1, EMB), _F32),
        ],
        scratch_shapes=[
            pltpu.VMEM((2 * CAT, EMB), _F32),
            pltpu.VMEM((128, EMB), _F32),
            pltpu.VMEM((1, EMB), _F32),
            pltpu.VMEM((1, EMB), _F32),
        ],
    )(features, pricei, wc2, wc3, we, bc2, bc3, be, ep_pad)


def _spmm_body(adj_ref, x_ref, t_ref, s_ref, acc_s):
    i = pl.program_id(0)

    @pl.when(i == 0)
    def _():
        acc_s[...] = jnp.zeros_like(acc_s)

    t = jnp.dot(adj_ref[...].astype(jnp.bfloat16), x_ref[...].astype(jnp.bfloat16),
                preferred_element_type=_F32)
    t_ref[...] = t
    acc_s[...] += jnp.sum(t, axis=0, keepdims=True)

    @pl.when(i == NB - 1)
    def _():
        s_ref[...] = acc_s[...]


def _spmm_call(adj, x):
    return pl.pallas_call(
        _spmm_body,
        grid=(NB,),
        in_specs=[
            pl.BlockSpec((RB, N), lambda i: (i, 0)),
            pl.BlockSpec((N, EMB), lambda i: (0, 0)),
        ],
        out_specs=[
            pl.BlockSpec((RB, EMB), lambda i: (i, 0)),
            pl.BlockSpec((1, EMB), lambda i: (0, 0)),
        ],
        out_shape=[
            jax.ShapeDtypeStruct((N, EMB), _F32),
            jax.ShapeDtypeStruct((1, EMB), _F32),
        ],
        scratch_shapes=[pltpu.VMEM((1, EMB), _F32)],
    )(adj, x)


W2 = 1024
NK2 = (N + W2 - 1) // W2
LASTW2 = N - (NK2 - 1) * W2
NPAD = NK2 * W2
RB2 = 2000
NG2 = N // RB2
GPB = RB2 // RB
_BF = jnp.bfloat16


def _p1_width(i2):
    return (RB2 * i2 // W2) * W2


def _p1_body(adj_ref, x_ref, t1_ref, t2p_ref, s1_ref, t1b_ref, s1_s):
    i = pl.program_id(0)
    acc1 = jnp.dot(adj_ref[...].astype(_BF), x_ref[...].astype(_BF),
                   preferred_element_type=_F32)
    t1_ref[...] = acc1
    t1b_ref[pl.ds(i * RB, RB), :] = acc1.astype(_BF)
    cs = jnp.sum(acc1, axis=0, keepdims=True)
    s1_s[...] = jnp.where(i == 0, cs, s1_s[...] + cs)

    @pl.when(i == NB - 1)
    def _():
        s1_ref[...] = s1_s[...]
        t1b_ref[N:NPAD, :] = jnp.zeros((NPAD - N, EMB), _BF)

    i2 = i // GPB

    @pl.when(i2 == 0)
    def _():
        t2p_ref[...] = jnp.zeros((RB, EMB), _F32)

    for gidx in range(1, NG2):
        wdt = _p1_width(gidx)

        @pl.when(i2 == gidx)
        def _():
            t2p_ref[...] = jnp.dot(adj_ref[:, 0:wdt].astype(_BF),
                                   t1b_ref[0:wdt, :],
                                   preferred_element_type=_F32)


def _p1_call(adj, x):
    return pl.pallas_call(
        _p1_body,
        grid=(NB,),
        in_specs=[
            pl.BlockSpec((RB, N), lambda i: (i, 0)),
            pl.BlockSpec((N, EMB), lambda i: (0, 0)),
        ],
        out_specs=[
            pl.BlockSpec((RB, EMB), lambda i: (i, 0)),
            pl.BlockSpec((RB, EMB), lambda i: (i, 0)),
            pl.BlockSpec((1, EMB), lambda i: (0, 0)),
            pl.BlockSpec((NPAD, EMB), lambda i: (0, 0)),
        ],
        out_shape=[
            jax.ShapeDtypeStruct((N, EMB), _F32),
            jax.ShapeDtypeStruct((N, EMB), _F32),
            jax.ShapeDtypeStruct((1, EMB), _F32),
            jax.ShapeDtypeStruct((NPAD, EMB), _BF),
        ],
        scratch_shapes=[
            pltpu.VMEM((1, EMB), _F32),
        ],
    )(adj, x)


def _build_p2_sched():
    steps = []
    for i2 in range(NG2):
        ks = list(range(_p1_width(i2) // W2, NK2))
        for n, k in enumerate(ks):
            fl = ((1 if n == 0 else 0)
                  | (2 if n == len(ks) - 1 else 0)
                  | (4 if k == NK2 - 1 else 0))
            steps.append((i2, k, fl))
    arr = np.asarray(steps, dtype=np.int32)
    return arr[:, 0], arr[:, 1], arr[:, 2]


_I2A, _K2A, _FL2 = _build_p2_sched()
_NP2 = len(_I2A)


def _p2_body(i2_ref, k_ref, fl_ref, adj_ref, t1p_ref, t2in_ref,
             t2_ref, s2_ref, acc_s, s2_s):
    g = pl.program_id(0)
    k = k_ref[g]
    fl = fl_ref[g]

    @pl.when(g == 0)
    def _():
        s2_s[...] = jnp.zeros_like(s2_s)

    @pl.when((fl & 1) != 0)
    def _():
        acc_s[...] = t2in_ref[...]

    @pl.when((fl & 4) == 0)
    def _():
        acc_s[...] += jnp.dot(adj_ref[...].astype(_BF),
                              t1p_ref[pl.ds(k * W2, W2), :],
                              preferred_element_type=_F32)

    @pl.when((fl & 4) != 0)
    def _():
        am = jnp.where(lax.broadcasted_iota(jnp.int32, (RB2, W2), 1) < LASTW2,
                       adj_ref[...], 0.0)
        acc_s[...] += jnp.dot(am.astype(_BF), t1p_ref[pl.ds(k * W2, W2), :],
                              preferred_element_type=_F32)

    @pl.when((fl & 2) != 0)
    def _():
        t2_ref[...] = acc_s[...]
        s2_s[...] += jnp.sum(acc_s[...], axis=0, keepdims=True)

    @pl.when(g == _NP2 - 1)
    def _():
        s2_ref[...] = s2_s[...]


def _p2_call(adj, t1pad, t2p):
    grid_spec = pltpu.PrefetchScalarGridSpec(
        num_scalar_prefetch=3,
        grid=(_NP2,),
        in_specs=[
            pl.BlockSpec((RB2, W2), lambda g, ia, ka, fl: (ia[g], ka[g])),
            pl.BlockSpec((NPAD, EMB), lambda g, ia, ka, fl: (0, 0)),
            pl.BlockSpec((RB2, EMB), lambda g, ia, ka, fl: (ia[g], 0)),
        ],
        out_specs=[
            pl.BlockSpec((RB2, EMB), lambda g, ia, ka, fl: (ia[g], 0)),
            pl.BlockSpec((1, EMB), lambda g, ia, ka, fl: (0, 0)),
        ],
        scratch_shapes=[
            pltpu.VMEM((RB2, EMB), _F32),
            pltpu.VMEM((1, EMB), _F32),
        ],
    )
    return pl.pallas_call(
        _p2_body,
        grid_spec=grid_spec,
        out_shape=[
            jax.ShapeDtypeStruct((N, EMB), _F32),
            jax.ShapeDtypeStruct((1, EMB), _F32),
        ],
    )(jnp.asarray(_I2A), jnp.asarray(_K2A), jnp.asarray(_FL2), adj, t1pad, t2p)


def _gc_body(t1_ref, t2_ref, it_ref, s1_ref, s2_ref, si_ref,
             wl_ref, wm_ref, g1_ref, be1_ref, g2_ref, be2_ref,
             wct_ref, wcb_ref, bcat_ref,
             a_ref, b_ref, c_ref, mu_s, glp_s, gmp_s):
    i = pl.program_id(0)
    ninv = 1.0 / N

    @pl.when(i == 0)
    def _():
        mu_s[0:1, :] = 0.5 * (s1_ref[...] + si_ref[...]) * ninv
        mu_s[1:2, :] = 0.5 * (s2_ref[...] - si_ref[...]) * ninv
        glp_s[...] = jnp.zeros_like(glp_s)
        gmp_s[...] = jnp.zeros_like(gmp_s)

    t1 = t1_ref[...]
    t2 = t2_ref[...]
    it = it_ref[...]
    lpc = 0.5 * (t1 + it) - mu_s[0:1, :]
    mpc = 0.5 * (t2 - it) - mu_s[1:2, :]
    dn = (((0,), (0,)), ((), ()))
    lpc = lpc.astype(jnp.bfloat16)
    mpc = mpc.astype(jnp.bfloat16)
    glp_s[...] += lax.dot_general(lpc, lpc, dn, preferred_element_type=_F32)
    gmp_s[...] += lax.dot_general(mpc, mpc, dn, preferred_element_type=_F32)

    @pl.when(i == NB - 1)
    def _():
        mu_lp = mu_s[0:1, :]
        mu_mp = mu_s[1:2, :]
        wl = wl_ref[...]
        wm = wm_ref[...]
        m1 = jnp.dot(mu_lp, wl, preferred_element_type=_F32)
        var1 = jnp.sum(jnp.dot(glp_s[...], wl, preferred_element_type=_F32) * wl,
                       axis=0, keepdims=True) * ninv
        a1 = g1_ref[...] / jnp.sqrt(var1 + 1e-5)
        m2 = jnp.dot(mu_mp, wm, preferred_element_type=_F32)
        var2 = jnp.sum(jnp.dot(gmp_s[...], wm, preferred_element_type=_F32) * wm,
                       axis=0, keepdims=True) * ninv
        a2 = g2_ref[...] / jnp.sqrt(var2 + 1e-5)
        wct = wct_ref[...]
        wcb = wcb_ref[...]
        a_ref[...] = 0.5 * jnp.dot(wl * a1, wct, preferred_element_type=_F32)
        b_ref[...] = 0.5 * jnp.dot(wm * a2, wcb, preferred_element_type=_F32)
        c_ref[...] = (jnp.dot(be1_ref[...] - m1 * a1, wct, preferred_element_type=_F32)
                      + jnp.dot(be2_ref[...] - m2 * a2, wcb, preferred_element_type=_F32)
                      + bcat_ref[...])


def _gc_call(t1, t2, item, s1, s2, si, wl, wm, g1, be1, g2, be2, wct, wcb, bcat):
    blk = pl.BlockSpec((RB, EMB), lambda i: (i, 0))
    vec = pl.BlockSpec((1, EMB), lambda i: (0, 0))
    small = pl.BlockSpec((EMB, EMB), lambda i: (0, 0))
    return pl.pallas_call(
        _gc_body,
        grid=(NB,),
        in_specs=[blk, blk, blk, vec, vec, vec,
                  small, small, vec, vec, vec, vec, small, small, vec],
        out_specs=[small, small, vec],
        out_shape=[
            jax.ShapeDtypeStruct((EMB, EMB), _F32),
            jax.ShapeDtypeStruct((EMB, EMB), _F32),
            jax.ShapeDtypeStruct((1, EMB), _F32),
        ],
        scratch_shapes=[
            pltpu.VMEM((2, EMB), _F32),
            pltpu.VMEM((EMB, EMB), _F32),
            pltpu.VMEM((EMB, EMB), _F32),
        ],
    )(t1, t2, item, s1, s2, si, wl, wm, g1, be1, g2, be2, wct, wcb, bcat)


_SC_INFO = plsc.get_sparse_core_info()
_NW = _SC_INFO.num_cores * _SC_INFO.num_subcores
_BPW = NGATHER // _NW
_CH = 96


def _make_sc_gather(ntab):
    def body(*refs):
        tables = refs[:ntab]
        idx_hbm = refs[ntab]
        outs = refs[ntab + 1:2 * ntab + 1]
        idx_v, rows_v, sem = refs[2 * ntab + 1:]
        wid = lax.axis_index("s") * _SC_INFO.num_cores + lax.axis_index("c")
        base = wid * _BPW
        for ci in range(_BPW // _CH):
            off = base + ci * _CH
            pltpu.sync_copy(idx_hbm.at[pl.ds(off, _CH)], idx_v)
            for tab, out in zip(tables, outs):
                pltpu.async_copy(tab.at[idx_v], rows_v, sem).wait()
                pltpu.sync_copy(rows_v, out.at[pl.ds(off, _CH)])

    return pl.kernel(
        body,
        mesh=plsc.VectorSubcoreMesh(core_axis_name="c", subcore_axis_name="s"),
        out_type=[jax.ShapeDtypeStruct((NGATHER, EMB), _F32)] * ntab,
        scratch_types=[
            pltpu.VMEM((_CH,), jnp.int32),
            pltpu.VMEM((_CH, EMB), _F32),
            pltpu.SemaphoreType.DMA,
        ],
    )


_sc_gather2 = _make_sc_gather(2)
_sc_gather1 = _make_sc_gather(1)


def _loss_body(t1g_ref, t2g_ref, ig_ref, a_ref, b_ref, c_ref, out_ref):
    ig = ig_ref[...]
    og = (jnp.dot(t1g_ref[...] + ig, a_ref[...], preferred_element_type=_F32)
          + jnp.dot(t2g_ref[...] - ig, b_ref[...], preferred_element_type=_F32)
          + c_ref[...])
    key = og[0:BSZ]
    pos = og[BSZ:2 * BSZ]
    ps = jnp.sum(key * pos, axis=1, keepdims=True)
    acc = jnp.zeros((1, 1), _F32)
    for k in range(NNEG):
        ns = jnp.sum(key * og[(2 + k) * BSZ:(3 + k) * BSZ], axis=1, keepdims=True)
        x = ps - ns
        sig = 1.0 / (1.0 + jnp.exp(-x))
        acc = acc + jnp.sum(jnp.log(sig + 1e-9))
    out_ref[...] = -acc / (BSZ * NNEG)


def _loss_call(t1g, t2g, ig, a, b, c):
    return pl.pallas_call(
        _loss_body,
        out_shape=jax.ShapeDtypeStruct((1, 1), _F32),
    )(t1g, t2g, ig, a, b, c)


def kernel(features, price, adj, train_set, W_cid2, b_cid2, W_cid3, b_cid3,
           emb_price, W_emb, b_emb, W_low, W_mid, g1, be1, g2, be2,
           W_cat, b_cat):
    pricei = price.reshape(N, 1)
    ep_pad = jnp.pad(emb_price, ((0, 128 - NBINS), (0, 0)))
    r = lambda v: v.reshape(1, EMB)

    item, s_it = _item_call(features, pricei, W_cid2, W_cid3, W_emb,
                            r(b_cid2), r(b_cid3), r(b_emb), ep_pad)
    t1, t2p, s1, t1b = _p1_call(adj, item)
    t2, s2 = _p2_call(adj, t1b, t2p)
    a, b, c = _gc_call(t1, t2, item, s1, s2, s_it, W_low, W_mid,
                       r(g1), r(be1), r(g2), r(be2),
                       W_cat[:EMB], W_cat[EMB:], r(b_cat))
    idx = train_set.T.reshape(-1)
    t1g, ig = _sc_gather2(t1, item, idx)
    t2g, = _sc_gather1(t2, idx)
    loss = _loss_call(t1g, t2g, ig, a, b, c)
    return loss.reshape(())
